# Initial kernel scaffold; baseline (speedup 1.0000x reference)
import jax
import jax.numpy as jnp
from jax.experimental import pallas as pl


def kernel(z, W_z, codebook, W_q):
    raise NotImplementedError("write your pallas kernel here")


# speedup vs baseline: 1.0689x; 1.0689x over previous
import functools

import jax
import jax.numpy as jnp
from jax.experimental import pallas as pl
from jax.experimental.pallas import tpu as pltpu

N_CODES = 1024
HIDDEN_DIM = 768
BOTTLENECK_DIM = 64
EPS = 1e-12


def _tc_body(z_ref, wz_ref, cb_ref, wq_ref, q_ref, codes_ref, loss_ref):
    i = pl.program_id(0)
    nsteps = pl.num_programs(0)

    @pl.when(i == 0)
    def _():
        loss_ref[0, 0] = 0.0

    h = jnp.dot(z_ref[...], wz_ref[...], preferred_element_type=jnp.float32)
    hnorm = jnp.sqrt(jnp.sum(h * h, axis=1, keepdims=True))
    h = h / jnp.maximum(hnorm, EPS)

    cb = cb_ref[...]
    cnorm = jnp.sqrt(jnp.sum(cb * cb, axis=1, keepdims=True))
    cn = cb / jnp.maximum(cnorm, EPS)

    zsq = jnp.sum(h * h, axis=1, keepdims=True)
    csq = jnp.sum(cn * cn, axis=1)[None, :]
    dist = zsq - 2.0 * jnp.dot(h, cn.T, preferred_element_type=jnp.float32) + csq

    dmin = jnp.min(dist, axis=1, keepdims=True)
    col = jax.lax.broadcasted_iota(jnp.int32, dist.shape, 1)
    codes = jnp.min(jnp.where(dist <= dmin, col, N_CODES), axis=1)
    codes_ref[0, 0, :] = codes

    onehot = (codes[:, None] == col).astype(jnp.float32)
    qn = jnp.dot(onehot, cn, preferred_element_type=jnp.float32)

    diff = qn - h
    blk = z_ref.shape[0]
    scale = 1.25 / (blk * nsteps * BOTTLENECK_DIM)
    loss_ref[0, 0] += jnp.sum(diff * diff) * scale

    q_ref[...] = jnp.dot(qn, wq_ref[...], preferred_element_type=jnp.float32)


@functools.partial(jax.jit, static_argnames=("blk",))
def _run(zf, W_z, codebook, W_q, blk=576):
    rows = zf.shape[0]
    nblk = rows // blk
    q, codes3, loss = pl.pallas_call(
        _tc_body,
        grid=(nblk,),
        in_specs=[
            pl.BlockSpec((blk, HIDDEN_DIM), lambda i: (i, 0)),
            pl.BlockSpec((HIDDEN_DIM, BOTTLENECK_DIM), lambda i: (0, 0)),
            pl.BlockSpec((N_CODES, BOTTLENECK_DIM), lambda i: (0, 0)),
            pl.BlockSpec((BOTTLENECK_DIM, HIDDEN_DIM), lambda i: (0, 0)),
        ],
        out_specs=[
            pl.BlockSpec((blk, HIDDEN_DIM), lambda i: (i, 0)),
            pl.BlockSpec((1, 1, blk), lambda i: (i, 0, 0)),
            pl.BlockSpec(memory_space=pltpu.SMEM),
        ],
        out_shape=[
            jax.ShapeDtypeStruct((rows, HIDDEN_DIM), jnp.float32),
            jax.ShapeDtypeStruct((nblk, 1, blk), jnp.int32),
            jax.ShapeDtypeStruct((1, 1), jnp.float32),
        ],
    )(zf, W_z, codebook, W_q)
    return q, codes3, loss


def kernel(z, W_z, codebook, W_q):
    B, T, D = z.shape
    zf = z.reshape(B * T, D)
    q, codes3, loss = _run(zf, W_z, codebook, W_q)
    return (q.reshape(B, T, D), codes3.reshape(B, T), loss[0, 0])
